# baseline (speedup 1.0000x reference)
import functools

import jax
import jax.numpy as jnp
from jax import lax
from jax.experimental import pallas as pl
from jax.experimental.pallas import tpu as pltpu
from jax.experimental.pallas import tpu_sc as plsc

_B, _N, _D, _K = 16, 576, 384, 5
_NEG = -3e38
_L = 16
_HALF = _N // 2
_GPW = _HALF // _L
_CH = 144
_NCH = _N // _CH
_WSTRIDE = 8 * _HALF


def _sim_body(x_ref, s_ref):
    x = x_ref[0]
    norm = jnp.sqrt(jnp.sum(x * x, axis=1, keepdims=True)) + 1e-8
    xn = x / norm
    S = lax.dot_general(xn, xn, (((1,), (1,)), ((), ())),
                        preferred_element_type=jnp.float32)
    row = lax.broadcasted_iota(jnp.int32, (_N, _N), 0)
    col = lax.broadcasted_iota(jnp.int32, (_N, _N), 1)
    s_ref[0] = jnp.where(row == col, _NEG, S)


def _sc_topk_body(s_hbm, idx_hbm, w_hbm, scol, vstate, istate, idxbuf, wbuf):
    b = lax.axis_index("s")
    half = lax.axis_index("c")
    nbase = half * _HALF
    negv = jnp.full((_L,), _NEG, jnp.float32)
    zeroi = jnp.zeros((_L,), jnp.int32)

    def chunk_body(c, carry):
        pltpu.sync_copy(s_hbm.at[b, pl.ds(c * _CH, _CH), :], scol)
        j0 = c * _CH

        def group_body(g, carry2):
            n0 = nbase + g * _L

            def scan1(jj, ts):
                t1, t2, t3, t4, t5, i1, i2, i3, i4, i5 = ts
                v = scol[jj, pl.ds(n0, _L)]
                i = zeroi + (j0 + jj)
                m = v > t1
                n1 = jnp.where(m, v, t1); j1 = jnp.where(m, i, i1)
                v = jnp.where(m, t1, v);  i = jnp.where(m, i1, i)
                m = v > t2
                n2 = jnp.where(m, v, t2); j2 = jnp.where(m, i, i2)
                v = jnp.where(m, t2, v);  i = jnp.where(m, i2, i)
                m = v > t3
                n3 = jnp.where(m, v, t3); j3 = jnp.where(m, i, i3)
                v = jnp.where(m, t3, v);  i = jnp.where(m, i3, i)
                m = v > t4
                n4 = jnp.where(m, v, t4); j4 = jnp.where(m, i, i4)
                v = jnp.where(m, t4, v);  i = jnp.where(m, i4, i)
                m = v > t5
                n5 = jnp.where(m, v, t5); j5 = jnp.where(m, i, i5)
                return (n1, n2, n3, n4, n5, j1, j2, j3, j4, j5)

            ts = tuple(vstate[k, g] for k in range(_K)) \
                + tuple(istate[k, g] for k in range(_K))
            out = lax.fori_loop(0, _CH, scan1, ts)
            for k in range(_K):
                vstate[k, g] = out[k]
                istate[k, g] = out[_K + k]
            return carry2

        lax.fori_loop(0, _GPW, group_body, 0)
        return carry

    def init_body(g, carry):
        for k in range(_K):
            vstate[k, g] = negv
            istate[k, g] = zeroi
        return carry

    lax.fori_loop(0, _GPW, init_body, 0)
    lax.fori_loop(0, _NCH, chunk_body, 0)

    def fin_body(g, carry):
        vals = [vstate[k, g] for k in range(_K)]
        e = [jnp.exp(vals[k] - vals[0]) for k in range(_K)]
        invd = 1.0 / (e[0] + e[1] + e[2] + e[3] + e[4])
        for k in range(_K):
            idxbuf[pl.ds(k * _HALF + g * _L, _L)] = istate[k, g]
            wbuf[pl.ds(k * _HALF + g * _L, _L)] = e[k] * invd
        return carry

    lax.fori_loop(0, _GPW, fin_body, 0)
    off = (b * 2 + half) * _WSTRIDE
    pltpu.sync_copy(idxbuf, idx_hbm.at[pl.ds(off, _WSTRIDE)])
    pltpu.sync_copy(wbuf, w_hbm.at[pl.ds(off, _WSTRIDE)])


_sc_topk = functools.partial(
    pl.kernel,
    out_type=(
        jax.ShapeDtypeStruct((2 * _B * _WSTRIDE,), jnp.int32),
        jax.ShapeDtypeStruct((2 * _B * _WSTRIDE,), jnp.float32),
    ),
    mesh=plsc.VectorSubcoreMesh(core_axis_name="c", subcore_axis_name="s"),
    scratch_types=[
        pltpu.VMEM((_CH, _N), jnp.float32),
        pltpu.VMEM((_K, _GPW, _L), jnp.float32),
        pltpu.VMEM((_K, _GPW, _L), jnp.int32),
        pltpu.VMEM((_WSTRIDE,), jnp.int32),
        pltpu.VMEM((_WSTRIDE,), jnp.float32),
    ],
)(_sc_topk_body)


def _mlp_body(x_ref, idx_ref, w_ref, w1_ref, b1_ref, w2_ref, b2_ref, out_ref):
    x = x_ref[0]
    idxv = idx_ref[0, 0]
    wv = w_ref[0, 0]
    rowi = lax.broadcasted_iota(jnp.int32, (_N, _HALF), 0)
    aggs = []
    for half in range(2):
        AT = jnp.zeros((_N, _HALF), jnp.float32)
        for k in range(_K):
            off = half * _WSTRIDE + k * _HALF
            ik = lax.slice(idxv, (off,), (off + _HALF,)).reshape(1, _HALF)
            wk = lax.slice(wv, (off,), (off + _HALF,)).reshape(1, _HALF)
            AT = AT + jnp.where(rowi == ik, wk, 0.0)
        aggs.append(lax.dot_general(AT, x, (((0,), (0,)), ((), ())),
                                    preferred_element_type=jnp.float32))
    agg = jnp.concatenate(aggs, axis=0)
    h = x + agg
    h1 = lax.dot_general(h, w1_ref[...], (((1,), (1,)), ((), ())),
                         preferred_element_type=jnp.float32)
    h1 = jnp.maximum(h1 + b1_ref[...], 0.0)
    h2 = lax.dot_general(h1, w2_ref[...], (((1,), (1,)), ((), ())),
                         preferred_element_type=jnp.float32)
    out_ref[0] = jnp.maximum(h2 + b2_ref[...], 0.0)


@jax.jit
def kernel(x, W1, b1, W2, b2):
    B, N, D = x.shape
    H = W1.shape[0]
    S = pl.pallas_call(
        _sim_body,
        grid=(B,),
        in_specs=[pl.BlockSpec((1, N, D), lambda b: (b, 0, 0))],
        out_specs=pl.BlockSpec((1, N, N), lambda b: (b, 0, 0)),
        out_shape=jax.ShapeDtypeStruct((B, N, N), jnp.float32),
        compiler_params=pltpu.CompilerParams(
            dimension_semantics=("arbitrary",),
        ),
    )(x)
    idx_flat, w_flat = _sc_topk(S)
    idx2 = idx_flat.reshape(B, 1, 2 * _WSTRIDE)
    w2 = w_flat.reshape(B, 1, 2 * _WSTRIDE)
    b1r = b1.reshape(1, H)
    b2r = b2.reshape(1, H)
    return pl.pallas_call(
        _mlp_body,
        grid=(B,),
        in_specs=[
            pl.BlockSpec((1, N, D), lambda b: (b, 0, 0)),
            pl.BlockSpec((1, 1, 2 * _WSTRIDE), lambda b: (b, 0, 0)),
            pl.BlockSpec((1, 1, 2 * _WSTRIDE), lambda b: (b, 0, 0)),
            pl.BlockSpec((H, D), lambda b: (0, 0)),
            pl.BlockSpec((1, H), lambda b: (0, 0)),
            pl.BlockSpec((H, H), lambda b: (0, 0)),
            pl.BlockSpec((1, H), lambda b: (0, 0)),
        ],
        out_specs=pl.BlockSpec((1, N, H), lambda b: (b, 0, 0)),
        out_shape=jax.ShapeDtypeStruct((B, N, H), jnp.float32),
        compiler_params=pltpu.CompilerParams(
            dimension_semantics=("arbitrary",),
        ),
    )(x, idx2, w2, W1, b1r, W2, b2r)


# speedup vs baseline: 1.1525x; 1.1525x over previous
import functools

import jax
import jax.numpy as jnp
from jax import lax
from jax.experimental import pallas as pl
from jax.experimental.pallas import tpu as pltpu
from jax.experimental.pallas import tpu_sc as plsc

_B, _N, _D, _K = 16, 576, 384, 5
_NEG = -3e38
_L = 16
_HALF = _N // 2
_GPW = _HALF // _L
_CH = 72
_NCH = _N // _CH
_IL = 3
_WSTRIDE = 8 * _HALF


def _sim_body(x_ref, s_ref):
    x = x_ref[0]
    norm = jnp.sqrt(jnp.sum(x * x, axis=1, keepdims=True)) + 1e-8
    xn = x / norm
    S = lax.dot_general(xn, xn, (((1,), (1,)), ((), ())),
                        preferred_element_type=jnp.float32)
    row = lax.broadcasted_iota(jnp.int32, (_N, _N), 0)
    col = lax.broadcasted_iota(jnp.int32, (_N, _N), 1)
    s_ref[0] = jnp.where(row == col, _NEG, S)


def _chain5(v, i, ts):
    t1, t2, t3, t4, t5, i1, i2, i3, i4, i5 = ts
    m = v > t1
    n1 = jnp.maximum(t1, v); j1 = jnp.where(m, i, i1)
    v = jnp.minimum(t1, v);  i = jnp.where(m, i1, i)
    m = v > t2
    n2 = jnp.maximum(t2, v); j2 = jnp.where(m, i, i2)
    v = jnp.minimum(t2, v);  i = jnp.where(m, i2, i)
    m = v > t3
    n3 = jnp.maximum(t3, v); j3 = jnp.where(m, i, i3)
    v = jnp.minimum(t3, v);  i = jnp.where(m, i3, i)
    m = v > t4
    n4 = jnp.maximum(t4, v); j4 = jnp.where(m, i, i4)
    v = jnp.minimum(t4, v);  i = jnp.where(m, i4, i)
    m = v > t5
    n5 = jnp.maximum(t5, v); j5 = jnp.where(m, i, i5)
    return (n1, n2, n3, n4, n5, j1, j2, j3, j4, j5)


def _sc_topk_body(s_hbm, idx_hbm, w_hbm, scol, vstate, istate, idxbuf, wbuf,
                  sem0, sem1):
    b = lax.axis_index("s")
    half = lax.axis_index("c")
    nbase = half * _HALF
    negv = jnp.full((_L,), _NEG, jnp.float32)
    zeroi = jnp.zeros((_L,), jnp.int32)

    def init_body(g, carry):
        for k in range(_K):
            vstate[k, g] = negv
            istate[k, g] = zeroi
        return carry

    lax.fori_loop(0, _GPW, init_body, 0)

    sems = [sem0, sem1]
    handles = [None, None]
    handles[0] = pltpu.async_copy(
        s_hbm.at[b, pl.ds(0, _CH), :], scol.at[0], sems[0])
    for c in range(_NCH):
        if c + 1 < _NCH:
            nxt = (c + 1) & 1
            handles[nxt] = pltpu.async_copy(
                s_hbm.at[b, pl.ds((c + 1) * _CH, _CH), :],
                scol.at[nxt], sems[nxt])
        handles[c & 1].wait()
        buf = c & 1
        j0 = c * _CH

        def tri_body(t, carry2, buf=buf, j0=j0):
            g = t * _IL
            ts = []
            for q in range(_IL):
                ts += [vstate[k, g + q] for k in range(_K)]
                ts += [istate[k, g + q] for k in range(_K)]

            def scan1(jj, s):
                i = zeroi + (j0 + jj)
                out = []
                for q in range(_IL):
                    n0 = nbase + (g + q) * _L
                    v = scol[buf, jj, pl.ds(n0, _L)]
                    out += list(_chain5(v, i, s[10 * q:10 * q + 10]))
                return tuple(out)

            out = lax.fori_loop(0, _CH, scan1, tuple(ts))
            for q in range(_IL):
                for k in range(_K):
                    vstate[k, g + q] = out[10 * q + k]
                    istate[k, g + q] = out[10 * q + _K + k]
            return carry2

        lax.fori_loop(0, _GPW // _IL, tri_body, 0)

    def fin_body(g, carry):
        vals = [vstate[k, g] for k in range(_K)]
        e = [jnp.exp(vals[k] - vals[0]) for k in range(_K)]
        invd = 1.0 / (e[0] + e[1] + e[2] + e[3] + e[4])
        for k in range(_K):
            idxbuf[pl.ds(k * _HALF + g * _L, _L)] = istate[k, g]
            wbuf[pl.ds(k * _HALF + g * _L, _L)] = e[k] * invd
        return carry

    lax.fori_loop(0, _GPW, fin_body, 0)
    off = (b * 2 + half) * _WSTRIDE
    pltpu.sync_copy(idxbuf, idx_hbm.at[pl.ds(off, _WSTRIDE)])
    pltpu.sync_copy(wbuf, w_hbm.at[pl.ds(off, _WSTRIDE)])


_sc_topk = functools.partial(
    pl.kernel,
    out_type=(
        jax.ShapeDtypeStruct((2 * _B * _WSTRIDE,), jnp.int32),
        jax.ShapeDtypeStruct((2 * _B * _WSTRIDE,), jnp.float32),
    ),
    mesh=plsc.VectorSubcoreMesh(core_axis_name="c", subcore_axis_name="s"),
    scratch_types=[
        pltpu.VMEM((2, _CH, _N), jnp.float32),
        pltpu.VMEM((_K, _GPW, _L), jnp.float32),
        pltpu.VMEM((_K, _GPW, _L), jnp.int32),
        pltpu.VMEM((_WSTRIDE,), jnp.int32),
        pltpu.VMEM((_WSTRIDE,), jnp.float32),
        pltpu.SemaphoreType.DMA,
        pltpu.SemaphoreType.DMA,
    ],
)(_sc_topk_body)


def _mlp_body(x_ref, idx_ref, w_ref, w1_ref, b1_ref, w2_ref, b2_ref, out_ref):
    x = x_ref[0]
    idxv = idx_ref[0, 0]
    wv = w_ref[0, 0]
    rowi = lax.broadcasted_iota(jnp.int32, (_N, _HALF), 0)
    aggs = []
    for half in range(2):
        AT = jnp.zeros((_N, _HALF), jnp.float32)
        for k in range(_K):
            off = half * _WSTRIDE + k * _HALF
            ik = lax.slice(idxv, (off,), (off + _HALF,)).reshape(1, _HALF)
            wk = lax.slice(wv, (off,), (off + _HALF,)).reshape(1, _HALF)
            AT = AT + jnp.where(rowi == ik, wk, 0.0)
        aggs.append(lax.dot_general(AT, x, (((0,), (0,)), ((), ())),
                                    preferred_element_type=jnp.float32))
    agg = jnp.concatenate(aggs, axis=0)
    h = x + agg
    h1 = lax.dot_general(h, w1_ref[...], (((1,), (1,)), ((), ())),
                         preferred_element_type=jnp.float32)
    h1 = jnp.maximum(h1 + b1_ref[...], 0.0)
    h2 = lax.dot_general(h1, w2_ref[...], (((1,), (1,)), ((), ())),
                         preferred_element_type=jnp.float32)
    out_ref[0] = jnp.maximum(h2 + b2_ref[...], 0.0)


@jax.jit
def kernel(x, W1, b1, W2, b2):
    B, N, D = x.shape
    H = W1.shape[0]
    S = pl.pallas_call(
        _sim_body,
        grid=(B,),
        in_specs=[pl.BlockSpec((1, N, D), lambda b: (b, 0, 0))],
        out_specs=pl.BlockSpec((1, N, N), lambda b: (b, 0, 0)),
        out_shape=jax.ShapeDtypeStruct((B, N, N), jnp.float32),
        compiler_params=pltpu.CompilerParams(
            dimension_semantics=("arbitrary",),
        ),
    )(x)
    idx_flat, w_flat = _sc_topk(S)
    idx2 = idx_flat.reshape(B, 1, 2 * _WSTRIDE)
    w2 = w_flat.reshape(B, 1, 2 * _WSTRIDE)
    b1r = b1.reshape(1, H)
    b2r = b2.reshape(1, H)
    return pl.pallas_call(
        _mlp_body,
        grid=(B,),
        in_specs=[
            pl.BlockSpec((1, N, D), lambda b: (b, 0, 0)),
            pl.BlockSpec((1, 1, 2 * _WSTRIDE), lambda b: (b, 0, 0)),
            pl.BlockSpec((1, 1, 2 * _WSTRIDE), lambda b: (b, 0, 0)),
            pl.BlockSpec((H, D), lambda b: (0, 0)),
            pl.BlockSpec((1, H), lambda b: (0, 0)),
            pl.BlockSpec((H, H), lambda b: (0, 0)),
            pl.BlockSpec((1, H), lambda b: (0, 0)),
        ],
        out_specs=pl.BlockSpec((1, N, H), lambda b: (b, 0, 0)),
        out_shape=jax.ShapeDtypeStruct((B, N, H), jnp.float32),
        compiler_params=pltpu.CompilerParams(
            dimension_semantics=("arbitrary",),
        ),
    )(x, idx2, w2, W1, b1r, W2, b2r)


# speedup vs baseline: 1.1711x; 1.0161x over previous
import functools

import jax
import jax.numpy as jnp
from jax import lax
from jax.experimental import pallas as pl
from jax.experimental.pallas import tpu as pltpu
from jax.experimental.pallas import tpu_sc as plsc

_B, _N, _D, _K = 16, 576, 384, 5
_NEG = -3e38
_L = 16
_HALF = _N // 2
_GPW = _HALF // _L
_CH = 72
_NCH = _N // _CH
_IL = 3
_NP = 640
_BW = 384
_BSH = 256
_WSTRIDE = 8 * _HALF


def _sim_body(x_ref, s_ref):
    x = x_ref[0]
    norm = jnp.sqrt(jnp.sum(x * x, axis=1, keepdims=True)) + 1e-8
    xn = x / norm
    S = lax.dot_general(xn, xn, (((1,), (1,)), ((), ())),
                        preferred_element_type=jnp.float32)
    row = lax.broadcasted_iota(jnp.int32, (_N, _N), 0)
    col = lax.broadcasted_iota(jnp.int32, (_N, _N), 1)
    S = jnp.where(row == col, _NEG, S)
    s_ref[0] = jnp.concatenate(
        [S, jnp.zeros((_N, _NP - _N), jnp.float32)], axis=1)


def _chain5(v, i, ts):
    t1, t2, t3, t4, t5, i1, i2, i3, i4, i5 = ts
    m = v > t1
    n1 = jnp.maximum(t1, v); j1 = jnp.where(m, i, i1)
    v = jnp.minimum(t1, v);  i = jnp.where(m, i1, i)
    m = v > t2
    n2 = jnp.maximum(t2, v); j2 = jnp.where(m, i, i2)
    v = jnp.minimum(t2, v);  i = jnp.where(m, i2, i)
    m = v > t3
    n3 = jnp.maximum(t3, v); j3 = jnp.where(m, i, i3)
    v = jnp.minimum(t3, v);  i = jnp.where(m, i3, i)
    m = v > t4
    n4 = jnp.maximum(t4, v); j4 = jnp.where(m, i, i4)
    v = jnp.minimum(t4, v);  i = jnp.where(m, i4, i)
    m = v > t5
    n5 = jnp.maximum(t5, v); j5 = jnp.where(m, i, i5)
    return (n1, n2, n3, n4, n5, j1, j2, j3, j4, j5)


def _sc_topk_body(s_hbm, idx_hbm, w_hbm, scol, vstate, istate, idxbuf, wbuf,
                  sem0, sem1):
    b = lax.axis_index("s")
    half = lax.axis_index("c")
    cb = half * _BSH
    nbase = half * _HALF - cb
    negv = jnp.full((_L,), _NEG, jnp.float32)
    zeroi = jnp.zeros((_L,), jnp.int32)

    def init_body(g, carry):
        for k in range(_K):
            vstate[k, g] = negv
            istate[k, g] = zeroi
        return carry

    lax.fori_loop(0, _GPW, init_body, 0)

    sems = [sem0, sem1]
    handles = [None, None]
    handles[0] = pltpu.async_copy(
        s_hbm.at[b, pl.ds(0, _CH), pl.ds(cb, _BW)], scol.at[0], sems[0])
    for c in range(_NCH):
        if c + 1 < _NCH:
            nxt = (c + 1) & 1
            handles[nxt] = pltpu.async_copy(
                s_hbm.at[b, pl.ds((c + 1) * _CH, _CH), pl.ds(cb, _BW)],
                scol.at[nxt], sems[nxt])
        handles[c & 1].wait()
        buf = c & 1
        j0 = c * _CH

        def tri_body(t, carry2, buf=buf, j0=j0):
            g = t * _IL
            ts = []
            for q in range(_IL):
                ts += [vstate[k, g + q] for k in range(_K)]
                ts += [istate[k, g + q] for k in range(_K)]

            def scan1(jj, s):
                i = zeroi + (j0 + jj)
                out = []
                for q in range(_IL):
                    n0 = nbase + (g + q) * _L
                    v = scol[buf, jj, pl.ds(n0, _L)]
                    out += list(_chain5(v, i, s[10 * q:10 * q + 10]))
                return tuple(out)

            out = lax.fori_loop(0, _CH, scan1, tuple(ts))
            for q in range(_IL):
                for k in range(_K):
                    vstate[k, g + q] = out[10 * q + k]
                    istate[k, g + q] = out[10 * q + _K + k]
            return carry2

        lax.fori_loop(0, _GPW // _IL, tri_body, 0)

    def fin_body(g, carry):
        vals = [vstate[k, g] for k in range(_K)]
        e = [jnp.exp(vals[k] - vals[0]) for k in range(_K)]
        invd = 1.0 / (e[0] + e[1] + e[2] + e[3] + e[4])
        for k in range(_K):
            idxbuf[pl.ds(k * _HALF + g * _L, _L)] = istate[k, g]
            wbuf[pl.ds(k * _HALF + g * _L, _L)] = e[k] * invd
        return carry

    lax.fori_loop(0, _GPW, fin_body, 0)
    off = (b * 2 + half) * _WSTRIDE
    pltpu.sync_copy(idxbuf, idx_hbm.at[pl.ds(off, _WSTRIDE)])
    pltpu.sync_copy(wbuf, w_hbm.at[pl.ds(off, _WSTRIDE)])


_sc_topk = functools.partial(
    pl.kernel,
    out_type=(
        jax.ShapeDtypeStruct((2 * _B * _WSTRIDE,), jnp.int32),
        jax.ShapeDtypeStruct((2 * _B * _WSTRIDE,), jnp.float32),
    ),
    mesh=plsc.VectorSubcoreMesh(core_axis_name="c", subcore_axis_name="s"),
    scratch_types=[
        pltpu.VMEM((2, _CH, _BW), jnp.float32),
        pltpu.VMEM((_K, _GPW, _L), jnp.float32),
        pltpu.VMEM((_K, _GPW, _L), jnp.int32),
        pltpu.VMEM((_WSTRIDE,), jnp.int32),
        pltpu.VMEM((_WSTRIDE,), jnp.float32),
        pltpu.SemaphoreType.DMA,
        pltpu.SemaphoreType.DMA,
    ],
)(_sc_topk_body)


def _mlp_body(x_ref, idx_ref, w_ref, w1_ref, b1_ref, w2_ref, b2_ref, out_ref):
    x = x_ref[0]
    idxv = idx_ref[0, 0]
    wv = w_ref[0, 0]
    rowi = lax.broadcasted_iota(jnp.int32, (_N, _HALF), 0)
    aggs = []
    for half in range(2):
        AT = jnp.zeros((_N, _HALF), jnp.float32)
        for k in range(_K):
            off = half * _WSTRIDE + k * _HALF
            ik = lax.slice(idxv, (off,), (off + _HALF,)).reshape(1, _HALF)
            wk = lax.slice(wv, (off,), (off + _HALF,)).reshape(1, _HALF)
            AT = AT + jnp.where(rowi == ik, wk, 0.0)
        aggs.append(lax.dot_general(AT, x, (((0,), (0,)), ((), ())),
                                    preferred_element_type=jnp.float32))
    agg = jnp.concatenate(aggs, axis=0)
    h = x + agg
    h1 = lax.dot_general(h, w1_ref[...], (((1,), (1,)), ((), ())),
                         preferred_element_type=jnp.float32)
    h1 = jnp.maximum(h1 + b1_ref[...], 0.0)
    h2 = lax.dot_general(h1, w2_ref[...], (((1,), (1,)), ((), ())),
                         preferred_element_type=jnp.float32)
    out_ref[0] = jnp.maximum(h2 + b2_ref[...], 0.0)


@jax.jit
def kernel(x, W1, b1, W2, b2):
    B, N, D = x.shape
    H = W1.shape[0]
    S = pl.pallas_call(
        _sim_body,
        grid=(B,),
        in_specs=[pl.BlockSpec((1, N, D), lambda b: (b, 0, 0))],
        out_specs=pl.BlockSpec((1, N, _NP), lambda b: (b, 0, 0)),
        out_shape=jax.ShapeDtypeStruct((B, N, _NP), jnp.float32),
        compiler_params=pltpu.CompilerParams(
            dimension_semantics=("arbitrary",),
        ),
    )(x)
    idx_flat, w_flat = _sc_topk(S)
    idx2 = idx_flat.reshape(B, 1, 2 * _WSTRIDE)
    w2 = w_flat.reshape(B, 1, 2 * _WSTRIDE)
    b1r = b1.reshape(1, H)
    b2r = b2.reshape(1, H)
    return pl.pallas_call(
        _mlp_body,
        grid=(B,),
        in_specs=[
            pl.BlockSpec((1, N, D), lambda b: (b, 0, 0)),
            pl.BlockSpec((1, 1, 2 * _WSTRIDE), lambda b: (b, 0, 0)),
            pl.BlockSpec((1, 1, 2 * _WSTRIDE), lambda b: (b, 0, 0)),
            pl.BlockSpec((H, D), lambda b: (0, 0)),
            pl.BlockSpec((1, H), lambda b: (0, 0)),
            pl.BlockSpec((H, H), lambda b: (0, 0)),
            pl.BlockSpec((1, H), lambda b: (0, 0)),
        ],
        out_specs=pl.BlockSpec((1, N, H), lambda b: (b, 0, 0)),
        out_shape=jax.ShapeDtypeStruct((B, N, H), jnp.float32),
        compiler_params=pltpu.CompilerParams(
            dimension_semantics=("arbitrary",),
        ),
    )(x, idx2, w2, W1, b1r, W2, b2r)


# speedup vs baseline: 1.1733x; 1.0019x over previous
import functools

import jax
import jax.numpy as jnp
from jax import lax
from jax.experimental import pallas as pl
from jax.experimental.pallas import tpu as pltpu
from jax.experimental.pallas import tpu_sc as plsc

_B, _N, _D, _K = 16, 576, 384, 5
_NEG = -3e38
_L = 16
_HALF = _N // 2
_GPW = _HALF // _L
_CH = 72
_NCH = _N // _CH
_IL = 3
_NP = 640
_BW = 384
_BSH = 256
_WSTRIDE = 8 * _HALF


def _sim_body(x_ref, s_ref):
    x = x_ref[0]
    norm = jnp.sqrt(jnp.sum(x * x, axis=1, keepdims=True)) + 1e-8
    xn = x / norm
    S = lax.dot_general(xn, xn, (((1,), (1,)), ((), ())),
                        preferred_element_type=jnp.float32)
    row = lax.broadcasted_iota(jnp.int32, (_N, _N), 0)
    col = lax.broadcasted_iota(jnp.int32, (_N, _N), 1)
    S = jnp.where(row == col, _NEG, S)
    s_ref[0] = jnp.concatenate(
        [S, jnp.zeros((_N, _NP - _N), jnp.float32)], axis=1)


def _chain5(v, i, ts):
    t1, t2, t3, t4, t5, i1, i2, i3, i4, i5 = ts
    m = v > t1
    n1 = jnp.maximum(t1, v); j1 = jnp.where(m, i, i1)
    v = jnp.minimum(t1, v);  i = jnp.where(m, i1, i)
    m = v > t2
    n2 = jnp.maximum(t2, v); j2 = jnp.where(m, i, i2)
    v = jnp.minimum(t2, v);  i = jnp.where(m, i2, i)
    m = v > t3
    n3 = jnp.maximum(t3, v); j3 = jnp.where(m, i, i3)
    v = jnp.minimum(t3, v);  i = jnp.where(m, i3, i)
    m = v > t4
    n4 = jnp.maximum(t4, v); j4 = jnp.where(m, i, i4)
    v = jnp.minimum(t4, v);  i = jnp.where(m, i4, i)
    m = v > t5
    n5 = jnp.maximum(t5, v); j5 = jnp.where(m, i, i5)
    return (n1, n2, n3, n4, n5, j1, j2, j3, j4, j5)


def _sc_topk_body(s_hbm, idx_hbm, w_hbm, scol, vstate, istate, idxbuf, wbuf,
                  sem0, sem1):
    b = lax.axis_index("s")
    half = lax.axis_index("c")
    cb = half * _BSH
    nbase = half * _HALF - cb
    negv = jnp.full((_L,), _NEG, jnp.float32)
    zeroi = jnp.zeros((_L,), jnp.int32)

    def init_body(g, carry):
        for k in range(_K):
            vstate[k, g] = negv
            istate[k, g] = zeroi
        return carry

    lax.fori_loop(0, _GPW, init_body, 0)

    sems = [sem0, sem1]
    handles = [None, None]
    handles[0] = pltpu.async_copy(
        s_hbm.at[b, pl.ds(0, _CH), pl.ds(cb, _BW)], scol.at[0], sems[0])
    for c in range(_NCH):
        if c + 1 < _NCH:
            nxt = (c + 1) & 1
            handles[nxt] = pltpu.async_copy(
                s_hbm.at[b, pl.ds((c + 1) * _CH, _CH), pl.ds(cb, _BW)],
                scol.at[nxt], sems[nxt])
        handles[c & 1].wait()
        buf = c & 1
        j0 = c * _CH

        def tri_body(t, carry2, buf=buf, j0=j0):
            g = t * _IL
            ts = []
            for q in range(_IL):
                ts += [vstate[k, g + q] for k in range(_K)]
                ts += [istate[k, g + q] for k in range(_K)]

            def scan1(jj, s):
                i = zeroi + (j0 + jj)
                out = []
                for q in range(_IL):
                    n0 = nbase + (g + q) * _L
                    v = scol[buf, jj, pl.ds(n0, _L)]
                    out += list(_chain5(v, i, s[10 * q:10 * q + 10]))
                return tuple(out)

            out = plsc.parallel_loop(0, _CH, carry=tuple(ts), unroll=2)(scan1)
            for q in range(_IL):
                for k in range(_K):
                    vstate[k, g + q] = out[10 * q + k]
                    istate[k, g + q] = out[10 * q + _K + k]
            return carry2

        lax.fori_loop(0, _GPW // _IL, tri_body, 0)

    def fin_body(g, carry):
        vals = [vstate[k, g] for k in range(_K)]
        e = [jnp.exp(vals[k] - vals[0]) for k in range(_K)]
        invd = 1.0 / (e[0] + e[1] + e[2] + e[3] + e[4])
        for k in range(_K):
            idxbuf[pl.ds(k * _HALF + g * _L, _L)] = istate[k, g]
            wbuf[pl.ds(k * _HALF + g * _L, _L)] = e[k] * invd
        return carry

    lax.fori_loop(0, _GPW, fin_body, 0)
    off = (b * 2 + half) * _WSTRIDE
    pltpu.sync_copy(idxbuf, idx_hbm.at[pl.ds(off, _WSTRIDE)])
    pltpu.sync_copy(wbuf, w_hbm.at[pl.ds(off, _WSTRIDE)])


_sc_topk = functools.partial(
    pl.kernel,
    out_type=(
        jax.ShapeDtypeStruct((2 * _B * _WSTRIDE,), jnp.int32),
        jax.ShapeDtypeStruct((2 * _B * _WSTRIDE,), jnp.float32),
    ),
    mesh=plsc.VectorSubcoreMesh(core_axis_name="c", subcore_axis_name="s"),
    scratch_types=[
        pltpu.VMEM((2, _CH, _BW), jnp.float32),
        pltpu.VMEM((_K, _GPW, _L), jnp.float32),
        pltpu.VMEM((_K, _GPW, _L), jnp.int32),
        pltpu.VMEM((_WSTRIDE,), jnp.int32),
        pltpu.VMEM((_WSTRIDE,), jnp.float32),
        pltpu.SemaphoreType.DMA,
        pltpu.SemaphoreType.DMA,
    ],
)(_sc_topk_body)


def _mlp_body(x_ref, idx_ref, w_ref, w1_ref, b1_ref, w2_ref, b2_ref, out_ref):
    x = x_ref[0]
    idxv = idx_ref[0, 0]
    wv = w_ref[0, 0]
    rowi = lax.broadcasted_iota(jnp.int32, (_N, _HALF), 0)
    aggs = []
    for half in range(2):
        AT = jnp.zeros((_N, _HALF), jnp.float32)
        for k in range(_K):
            off = half * _WSTRIDE + k * _HALF
            ik = lax.slice(idxv, (off,), (off + _HALF,)).reshape(1, _HALF)
            wk = lax.slice(wv, (off,), (off + _HALF,)).reshape(1, _HALF)
            AT = AT + jnp.where(rowi == ik, wk, 0.0)
        aggs.append(lax.dot_general(AT, x, (((0,), (0,)), ((), ())),
                                    preferred_element_type=jnp.float32))
    agg = jnp.concatenate(aggs, axis=0)
    h = x + agg
    h1 = lax.dot_general(h, w1_ref[...], (((1,), (1,)), ((), ())),
                         preferred_element_type=jnp.float32)
    h1 = jnp.maximum(h1 + b1_ref[...], 0.0)
    h2 = lax.dot_general(h1, w2_ref[...], (((1,), (1,)), ((), ())),
                         preferred_element_type=jnp.float32)
    out_ref[0] = jnp.maximum(h2 + b2_ref[...], 0.0)


@jax.jit
def kernel(x, W1, b1, W2, b2):
    B, N, D = x.shape
    H = W1.shape[0]
    S = pl.pallas_call(
        _sim_body,
        grid=(B,),
        in_specs=[pl.BlockSpec((1, N, D), lambda b: (b, 0, 0))],
        out_specs=pl.BlockSpec((1, N, _NP), lambda b: (b, 0, 0)),
        out_shape=jax.ShapeDtypeStruct((B, N, _NP), jnp.float32),
        compiler_params=pltpu.CompilerParams(
            dimension_semantics=("arbitrary",),
        ),
    )(x)
    idx_flat, w_flat = _sc_topk(S)
    idx2 = idx_flat.reshape(B, 1, 2 * _WSTRIDE)
    w2 = w_flat.reshape(B, 1, 2 * _WSTRIDE)
    b1r = b1.reshape(1, H)
    b2r = b2.reshape(1, H)
    return pl.pallas_call(
        _mlp_body,
        grid=(B,),
        in_specs=[
            pl.BlockSpec((1, N, D), lambda b: (b, 0, 0)),
            pl.BlockSpec((1, 1, 2 * _WSTRIDE), lambda b: (b, 0, 0)),
            pl.BlockSpec((1, 1, 2 * _WSTRIDE), lambda b: (b, 0, 0)),
            pl.BlockSpec((H, D), lambda b: (0, 0)),
            pl.BlockSpec((1, H), lambda b: (0, 0)),
            pl.BlockSpec((H, H), lambda b: (0, 0)),
            pl.BlockSpec((1, H), lambda b: (0, 0)),
        ],
        out_specs=pl.BlockSpec((1, N, H), lambda b: (b, 0, 0)),
        out_shape=jax.ShapeDtypeStruct((B, N, H), jnp.float32),
        compiler_params=pltpu.CompilerParams(
            dimension_semantics=("arbitrary",),
        ),
    )(x, idx2, w2, W1, b1r, W2, b2r)


# speedup vs baseline: 1.2028x; 1.0252x over previous
import functools

import jax
import jax.numpy as jnp
from jax import lax
from jax.experimental import pallas as pl
from jax.experimental.pallas import tpu as pltpu
from jax.experimental.pallas import tpu_sc as plsc

_B, _N, _D, _K = 16, 576, 384, 5
_NEG = -3e38
_L = 16
_HALF = _N // 2
_GPW = _HALF // _L
_CH = 72
_NCH = _N // _CH
_IL = 3
_NP = 640
_BW = 384
_BSH = 256
_WSTRIDE = 8 * _HALF


def _sim_body(x_ref, s_ref):
    x = x_ref[0]
    norm = jnp.sqrt(jnp.sum(x * x, axis=1, keepdims=True)) + 1e-8
    xn = x / norm
    S = lax.dot_general(xn, xn, (((1,), (1,)), ((), ())),
                        preferred_element_type=jnp.float32)
    row = lax.broadcasted_iota(jnp.int32, (_N, _N), 0)
    col = lax.broadcasted_iota(jnp.int32, (_N, _N), 1)
    S = jnp.where(row == col, -1.5, S)
    s_ref[0] = jnp.concatenate(
        [S, jnp.zeros((_N, _NP - _N), jnp.float32)], axis=1)


def _chain5(k, ts):
    t1, t2, t3, t4, t5 = ts
    n1 = jnp.maximum(t1, k); k = jnp.minimum(t1, k)
    n2 = jnp.maximum(t2, k); k = jnp.minimum(t2, k)
    n3 = jnp.maximum(t3, k); k = jnp.minimum(t3, k)
    n4 = jnp.maximum(t4, k); k = jnp.minimum(t4, k)
    n5 = jnp.maximum(t5, k)
    return (n1, n2, n3, n4, n5)


def _sc_topk_body(s_hbm, idx_hbm, w_hbm, scol, kstate, idxbuf, wbuf,
                  sem0, sem1):
    b = lax.axis_index("s")
    half = lax.axis_index("c")
    cb = half * _BSH
    nbase = half * _HALF - cb
    neg = jnp.full((_L,), jnp.int32(-0x7FFF0000), jnp.int32)
    zeroi = jnp.zeros((_L,), jnp.int32)
    ten = jnp.full((_L,), 10, jnp.int32)

    def init_body(g, carry):
        for k in range(_K):
            kstate[k, g] = neg
        return carry

    lax.fori_loop(0, _GPW, init_body, 0)

    sems = [sem0, sem1]
    handles = [None, None]
    handles[0] = pltpu.async_copy(
        s_hbm.at[b, pl.ds(0, _CH), pl.ds(cb, _BW)], scol.at[0], sems[0])
    for c in range(_NCH):
        if c + 1 < _NCH:
            nxt = (c + 1) & 1
            handles[nxt] = pltpu.async_copy(
                s_hbm.at[b, pl.ds((c + 1) * _CH, _CH), pl.ds(cb, _BW)],
                scol.at[nxt], sems[nxt])
        handles[c & 1].wait()
        buf = c & 1
        j0 = c * _CH

        def tri_body(t, carry2, buf=buf, j0=j0):
            g = t * _IL
            ts = []
            for q in range(_IL):
                ts += [kstate[k, g + q] for k in range(_K)]

            def scan1(jj, s):
                i = zeroi + (1023 - j0 - jj)
                out = []
                for q in range(_IL):
                    n0 = nbase + (g + q) * _L
                    v = scol[buf, jj, pl.ds(n0, _L)]
                    vi = lax.convert_element_type(v * 1048576.0, jnp.int32)
                    key = lax.shift_left(vi, ten) | i
                    out += list(_chain5(key, s[_K * q:_K * q + _K]))
                return tuple(out)

            out = plsc.parallel_loop(0, _CH, carry=tuple(ts), unroll=2)(scan1)
            for q in range(_IL):
                for k in range(_K):
                    kstate[k, g + q] = out[_K * q + k]
            return carry2

        lax.fori_loop(0, _GPW // _IL, tri_body, 0)

    def fin_body(g, carry):
        keys = [kstate[k, g] for k in range(_K)]
        vis = [lax.shift_right_arithmetic(keys[k], ten) for k in range(_K)]
        e = [jnp.exp(lax.convert_element_type(vis[k] - vis[0], jnp.float32)
                     * 9.5367431640625e-07) for k in range(_K)]
        invd = 1.0 / (e[0] + e[1] + e[2] + e[3] + e[4])
        ji = jnp.full((_L,), 1023, jnp.int32)
        for k in range(_K):
            idxbuf[pl.ds(k * _HALF + g * _L, _L)] = ji - (keys[k] & ji)
            wbuf[pl.ds(k * _HALF + g * _L, _L)] = e[k] * invd
        return carry

    lax.fori_loop(0, _GPW, fin_body, 0)
    off = (b * 2 + half) * _WSTRIDE
    pltpu.sync_copy(idxbuf, idx_hbm.at[pl.ds(off, _WSTRIDE)])
    pltpu.sync_copy(wbuf, w_hbm.at[pl.ds(off, _WSTRIDE)])


_sc_topk = functools.partial(
    pl.kernel,
    out_type=(
        jax.ShapeDtypeStruct((2 * _B * _WSTRIDE,), jnp.int32),
        jax.ShapeDtypeStruct((2 * _B * _WSTRIDE,), jnp.float32),
    ),
    mesh=plsc.VectorSubcoreMesh(core_axis_name="c", subcore_axis_name="s"),
    scratch_types=[
        pltpu.VMEM((2, _CH, _BW), jnp.float32),
        pltpu.VMEM((_K, _GPW, _L), jnp.int32),
        pltpu.VMEM((_WSTRIDE,), jnp.int32),
        pltpu.VMEM((_WSTRIDE,), jnp.float32),
        pltpu.SemaphoreType.DMA,
        pltpu.SemaphoreType.DMA,
    ],
)(_sc_topk_body)


def _mlp_body(x_ref, idx_ref, w_ref, w1_ref, b1_ref, w2_ref, b2_ref, out_ref):
    x = x_ref[0]
    idxv = idx_ref[0, 0]
    wv = w_ref[0, 0]
    rowi = lax.broadcasted_iota(jnp.int32, (_N, _HALF), 0)
    aggs = []
    for half in range(2):
        AT = jnp.zeros((_N, _HALF), jnp.float32)
        for k in range(_K):
            off = half * _WSTRIDE + k * _HALF
            ik = lax.slice(idxv, (off,), (off + _HALF,)).reshape(1, _HALF)
            wk = lax.slice(wv, (off,), (off + _HALF,)).reshape(1, _HALF)
            AT = AT + jnp.where(rowi == ik, wk, 0.0)
        aggs.append(lax.dot_general(AT, x, (((0,), (0,)), ((), ())),
                                    preferred_element_type=jnp.float32))
    agg = jnp.concatenate(aggs, axis=0)
    h = x + agg
    h1 = lax.dot_general(h, w1_ref[...], (((1,), (1,)), ((), ())),
                         preferred_element_type=jnp.float32)
    h1 = jnp.maximum(h1 + b1_ref[...], 0.0)
    h2 = lax.dot_general(h1, w2_ref[...], (((1,), (1,)), ((), ())),
                         preferred_element_type=jnp.float32)
    out_ref[0] = jnp.maximum(h2 + b2_ref[...], 0.0)


@jax.jit
def kernel(x, W1, b1, W2, b2):
    B, N, D = x.shape
    H = W1.shape[0]
    S = pl.pallas_call(
        _sim_body,
        grid=(B,),
        in_specs=[pl.BlockSpec((1, N, D), lambda b: (b, 0, 0))],
        out_specs=pl.BlockSpec((1, N, _NP), lambda b: (b, 0, 0)),
        out_shape=jax.ShapeDtypeStruct((B, N, _NP), jnp.float32),
        compiler_params=pltpu.CompilerParams(
            dimension_semantics=("arbitrary",),
        ),
    )(x)
    idx_flat, w_flat = _sc_topk(S)
    idx2 = idx_flat.reshape(B, 1, 2 * _WSTRIDE)
    w2 = w_flat.reshape(B, 1, 2 * _WSTRIDE)
    b1r = b1.reshape(1, H)
    b2r = b2.reshape(1, H)
    return pl.pallas_call(
        _mlp_body,
        grid=(B,),
        in_specs=[
            pl.BlockSpec((1, N, D), lambda b: (b, 0, 0)),
            pl.BlockSpec((1, 1, 2 * _WSTRIDE), lambda b: (b, 0, 0)),
            pl.BlockSpec((1, 1, 2 * _WSTRIDE), lambda b: (b, 0, 0)),
            pl.BlockSpec((H, D), lambda b: (0, 0)),
            pl.BlockSpec((1, H), lambda b: (0, 0)),
            pl.BlockSpec((H, H), lambda b: (0, 0)),
            pl.BlockSpec((1, H), lambda b: (0, 0)),
        ],
        out_specs=pl.BlockSpec((1, N, H), lambda b: (b, 0, 0)),
        out_shape=jax.ShapeDtypeStruct((B, N, H), jnp.float32),
        compiler_params=pltpu.CompilerParams(
            dimension_semantics=("arbitrary",),
        ),
    )(x, idx2, w2, W1, b1r, W2, b2r)
